# baseline (speedup 1.0000x reference)
import jax
import jax.numpy as jnp
from jax.experimental import pallas as pl
from jax.experimental.pallas import tpu as pltpu

D = 768
E = 64
TILE = 1024


def _router_kernel(xm_ref, xs_ref, wa_ref, ba_ref, ws_ref, bs_ref, oa_ref, os_ref):
    la = jnp.dot(xm_ref[:], wa_ref[:], preferred_element_type=jnp.float32) + ba_ref[:]
    ls = jnp.dot(xs_ref[:], ws_ref[:], preferred_element_type=jnp.float32) + bs_ref[:]
    ma = jnp.max(la, axis=-1, keepdims=True)
    ea = jnp.exp(la - ma)
    oa_ref[:] = (ea / jnp.sum(ea, axis=-1, keepdims=True)).T
    ms = jnp.max(ls, axis=-1, keepdims=True)
    es = jnp.exp(ls - ms)
    os_ref[:] = (es / jnp.sum(es, axis=-1, keepdims=True)).T


def kernel(x_m, x_s, W_a, b_a, W_s, b_s):
    n = x_m.shape[0]
    ba = b_a.reshape(1, E)
    bs = b_s.reshape(1, E)
    out = pl.pallas_call(
        _router_kernel,
        grid=(n // TILE,),
        in_specs=[
            pl.BlockSpec((TILE, D), lambda i: (i, 0)),
            pl.BlockSpec((TILE, D), lambda i: (i, 0)),
            pl.BlockSpec((D, E), lambda i: (0, 0)),
            pl.BlockSpec((1, E), lambda i: (0, 0)),
            pl.BlockSpec((D, E), lambda i: (0, 0)),
            pl.BlockSpec((1, E), lambda i: (0, 0)),
        ],
        out_specs=[
            pl.BlockSpec((E, TILE), lambda i: (0, i)),
            pl.BlockSpec((E, TILE), lambda i: (0, i)),
        ],
        out_shape=[
            jax.ShapeDtypeStruct((E, n), jnp.float32),
            jax.ShapeDtypeStruct((E, n), jnp.float32),
        ],
        compiler_params=pltpu.CompilerParams(
            dimension_semantics=("parallel",),
        ),
    )(x_m, x_s, W_a, ba, W_s, bs)
    return (out[0].T, out[1].T)


# speedup vs baseline: 1.1716x; 1.1716x over previous
import jax
import jax.numpy as jnp
from jax.experimental import pallas as pl
from jax.experimental.pallas import tpu as pltpu

D = 768
E = 64
TILE = 2048


def _router_kernel(xm_ref, xs_ref, wa_ref, ba_ref, ws_ref, bs_ref, oa_ref, os_ref):
    dn = (((1,), (1,)), ((), ()))
    la = jax.lax.dot_general(xm_ref[:], wa_ref[:], dn,
                             preferred_element_type=jnp.float32) + ba_ref[:]
    ls = jax.lax.dot_general(xs_ref[:], ws_ref[:], dn,
                             preferred_element_type=jnp.float32) + bs_ref[:]
    ma = jnp.max(la, axis=-1, keepdims=True)
    ea = jnp.exp(la - ma)
    oa_ref[:] = (ea / jnp.sum(ea, axis=-1, keepdims=True)).T
    ms = jnp.max(ls, axis=-1, keepdims=True)
    es = jnp.exp(ls - ms)
    os_ref[:] = (es / jnp.sum(es, axis=-1, keepdims=True)).T


def kernel(x_m, x_s, W_a, b_a, W_s, b_s):
    n = x_m.shape[0]
    ba = b_a.reshape(1, E)
    bs = b_s.reshape(1, E)
    out = pl.pallas_call(
        _router_kernel,
        grid=(n // TILE,),
        in_specs=[
            pl.BlockSpec((TILE, D), lambda i: (i, 0)),
            pl.BlockSpec((TILE, D), lambda i: (i, 0)),
            pl.BlockSpec((E, D), lambda i: (0, 0)),
            pl.BlockSpec((1, E), lambda i: (0, 0)),
            pl.BlockSpec((E, D), lambda i: (0, 0)),
            pl.BlockSpec((1, E), lambda i: (0, 0)),
        ],
        out_specs=[
            pl.BlockSpec((E, TILE), lambda i: (0, i)),
            pl.BlockSpec((E, TILE), lambda i: (0, i)),
        ],
        out_shape=[
            jax.ShapeDtypeStruct((E, n), jnp.float32),
            jax.ShapeDtypeStruct((E, n), jnp.float32),
        ],
        compiler_params=pltpu.CompilerParams(
            dimension_semantics=("parallel",),
        ),
    )(x_m, x_s, W_a.T, ba, W_s.T, bs)
    return (out[0].T, out[1].T)


# speedup vs baseline: 1.1872x; 1.0133x over previous
import jax
import jax.numpy as jnp
from jax.experimental import pallas as pl
from jax.experimental.pallas import tpu as pltpu

D = 768
E = 64
TILE = 4096


def _router_kernel(xm_ref, xs_ref, wa_ref, ba_ref, ws_ref, bs_ref, oa_ref, os_ref):
    dn = (((1,), (1,)), ((), ()))
    la = jax.lax.dot_general(xm_ref[:], wa_ref[:], dn,
                             preferred_element_type=jnp.float32) + ba_ref[:]
    ls = jax.lax.dot_general(xs_ref[:], ws_ref[:], dn,
                             preferred_element_type=jnp.float32) + bs_ref[:]
    ma = jnp.max(la, axis=-1, keepdims=True)
    ea = jnp.exp(la - ma)
    oa_ref[:] = (ea / jnp.sum(ea, axis=-1, keepdims=True)).T
    ms = jnp.max(ls, axis=-1, keepdims=True)
    es = jnp.exp(ls - ms)
    os_ref[:] = (es / jnp.sum(es, axis=-1, keepdims=True)).T


def kernel(x_m, x_s, W_a, b_a, W_s, b_s):
    n = x_m.shape[0]
    ba = b_a.reshape(1, E)
    bs = b_s.reshape(1, E)
    out = pl.pallas_call(
        _router_kernel,
        grid=(n // TILE,),
        in_specs=[
            pl.BlockSpec((TILE, D), lambda i: (i, 0)),
            pl.BlockSpec((TILE, D), lambda i: (i, 0)),
            pl.BlockSpec((E, D), lambda i: (0, 0)),
            pl.BlockSpec((1, E), lambda i: (0, 0)),
            pl.BlockSpec((E, D), lambda i: (0, 0)),
            pl.BlockSpec((1, E), lambda i: (0, 0)),
        ],
        out_specs=[
            pl.BlockSpec((E, TILE), lambda i: (0, i)),
            pl.BlockSpec((E, TILE), lambda i: (0, i)),
        ],
        out_shape=[
            jax.ShapeDtypeStruct((E, n), jnp.float32),
            jax.ShapeDtypeStruct((E, n), jnp.float32),
        ],
        compiler_params=pltpu.CompilerParams(
            dimension_semantics=("parallel",),
            vmem_limit_bytes=110 * 1024 * 1024,
        ),
    )(x_m, x_s, W_a.T, ba, W_s.T, bs)
    return (out[0].T, out[1].T)
